# Initial kernel scaffold; baseline (speedup 1.0000x reference)
import jax
import jax.numpy as jnp
from jax.experimental import pallas as pl


def kernel(x, m1_0, m1_1, m1_2, c1_0, c1_2, b1s, re1, ro1, pe1, po1, m2_0, m2_1, m2_2, c2_0, c2_2, b2s, re2, ro2, pe2, po2, fc1_w, fc1_b, fc2_w, fc2_b, fc3_w, fc3_b):
    raise NotImplementedError("write your pallas kernel here")


# speedup vs baseline: 2.7481x; 2.7481x over previous
import numpy as np
import jax
import jax.numpy as jnp
from jax.experimental import pallas as pl
from jax.experimental.pallas import tpu as pltpu

_PACK = 8


def _round_up(n, m):
    return ((n + m - 1) // m) * m


def _col_compact_sel(w, pack):
    S = np.zeros((w, w // 2), np.float32)
    S[2 * np.arange(w // 2), np.arange(w // 2)] = 1.0
    return np.kron(np.eye(pack, dtype=np.float32), S)


def _shift_lr(x, img_w):
    R, L = x.shape
    z = jnp.zeros((R, 1), x.dtype)
    xl = jnp.concatenate([x[:, 1:], z], axis=1)
    xr = jnp.concatenate([z, x[:, :-1]], axis=1)
    col = jax.lax.broadcasted_iota(jnp.int32, (1, L), 1) % img_w
    xl = jnp.where(col == img_w - 1, jnp.zeros((), x.dtype), xl)
    xr = jnp.where(col == 0, jnp.zeros((), x.dtype), xr)
    return xr, xl


def _pool2x2(y, scratches, sel_ref):
    R, L = y.shape
    t = jnp.maximum(y, jnp.concatenate([y[1:, :], y[:1, :]], axis=0))
    nc = L // 128
    for c in range(nc):
        scratches[c][:R, :] = t[:, c * 128:(c + 1) * 128]
    tr = jnp.concatenate(
        [scratches[c][pl.ds(0, R // 2, 2), :] for c in range(nc)], axis=1)
    u = jnp.maximum(tr, jnp.concatenate([tr[:, 1:], tr[:, :1]], axis=1))
    return jnp.dot(u.astype(sel_ref.dtype), sel_ref[:L, :],
                   preferred_element_type=jnp.float32)


def _conv_stack_kernel(x_ref, m1_ref, b1_ref, m2_ref, b2_ref,
                       s1_ref, s2_ref, out_ref, *scratches):
    f32 = jnp.float32
    bf16 = jnp.bfloat16

    x = x_ref[0, :, :]
    xr, xl = _shift_lr(x, 64)
    xs = jnp.concatenate([xr, x, xl], axis=0).astype(bf16)

    y = jnp.dot(m1_ref[...], xs, preferred_element_type=f32)
    y = jnp.maximum(y + b1_ref[...], 0.0)
    p1 = _pool2x2(y, scratches, s1_ref)

    p1r, p1l = _shift_lr(p1, 32)
    ps = jnp.concatenate([p1r, p1, p1l], axis=0).astype(bf16)

    y2 = jnp.dot(m2_ref[...], ps, preferred_element_type=f32)
    y2 = jnp.maximum(y2 + b2_ref[...], 0.0)
    p2 = _pool2x2(y2, scratches, s2_ref)

    out_ref[0, :, :] = p2.astype(out_ref.dtype)


def _fc_stack_kernel(x_ref, w1_ref, b1_ref, w2_ref, b2_ref, w3_ref, b3_ref,
                     o_ref):
    f32 = jnp.float32
    h = jnp.dot(x_ref[...], w1_ref[...], preferred_element_type=f32)
    h = jnp.maximum(h + b1_ref[...], 0.0)
    h = jnp.dot(h, w2_ref[...], preferred_element_type=f32)
    h = jnp.maximum(h + b2_ref[...], 0.0)
    o = jnp.dot(h, w3_ref[...], preferred_element_type=f32) + b3_ref[...]
    o_ref[...] = o.astype(o_ref.dtype)


def kernel(x, m1_0, m1_1, m1_2, c1_0, c1_2, b1s, re1, ro1, pe1, po1,
           m2_0, m2_1, m2_2, c2_0, c2_2, b2s, re2, ro2, pe2, po2,
           fc1_w, fc1_b, fc2_w, fc2_b, fc3_w, fc3_b):
    f32 = jnp.float32
    bf16 = jnp.bfloat16

    N = x.shape[0]
    assert x.shape[1:] == (3, 64, 64), x.shape
    Np = _round_up(N, _PACK)
    x = x.astype(f32)
    if Np != N:
        x = jnp.pad(x, ((0, Np - N), (0, 0), (0, 0), (0, 0)))
    Nb = Np // _PACK

    xp = x.reshape(Nb, _PACK, 3, 64, 64).transpose(0, 2, 3, 1, 4)
    xp = xp.reshape(Nb, 3 * 64, _PACK * 64)

    m1 = jnp.concatenate([m1_0, m1_1, m1_2], axis=1).astype(bf16)
    m2 = jnp.concatenate([m2_0, m2_1, m2_2], axis=1).astype(bf16)
    s1 = jnp.asarray(_col_compact_sel(64, _PACK), bf16)
    s2 = jnp.asarray(_col_compact_sel(32, _PACK), bf16)

    conv_out = pl.pallas_call(
        _conv_stack_kernel,
        out_shape=jax.ShapeDtypeStruct((Nb, 12 * 16, _PACK * 16), bf16),
        grid=(Nb,),
        in_specs=[
            pl.BlockSpec((1, 3 * 64, _PACK * 64), lambda i: (i, 0, 0)),
            pl.BlockSpec(m1.shape, lambda i: (0, 0)),
            pl.BlockSpec(b1s.shape, lambda i: (0, 0)),
            pl.BlockSpec(m2.shape, lambda i: (0, 0)),
            pl.BlockSpec(b2s.shape, lambda i: (0, 0)),
            pl.BlockSpec(s1.shape, lambda i: (0, 0)),
            pl.BlockSpec(s2.shape, lambda i: (0, 0)),
        ],
        out_specs=pl.BlockSpec((1, 12 * 16, _PACK * 16), lambda i: (i, 0, 0)),
        scratch_shapes=[pltpu.VMEM((6 * 64, 128), f32)
                        for _ in range(_PACK * 64 // 128)],
        compiler_params=pltpu.CompilerParams(dimension_semantics=("parallel",)),
    )(xp, m1, b1s.astype(f32), m2, b2s.astype(f32), s1, s2)

    feat = conv_out.reshape(Nb, 12, 16, _PACK, 16).transpose(0, 3, 1, 2, 4)
    flat = feat.reshape(Np, 12 * 16 * 16)[:N]

    K = flat.shape[1]
    n1 = fc1_w.shape[1]
    n2 = fc2_w.shape[1]
    n3 = fc3_w.shape[1]

    TB = min(128, _round_up(N, 8))
    Nf = _round_up(N, TB)
    if Nf != N:
        flat = jnp.pad(flat, ((0, Nf - N), (0, 0)))

    out = pl.pallas_call(
        _fc_stack_kernel,
        out_shape=jax.ShapeDtypeStruct((Nf, n3), f32),
        grid=(Nf // TB,),
        in_specs=[
            pl.BlockSpec((TB, K), lambda i: (i, 0)),
            pl.BlockSpec((K, n1), lambda i: (0, 0)),
            pl.BlockSpec((1, n1), lambda i: (0, 0)),
            pl.BlockSpec((n1, n2), lambda i: (0, 0)),
            pl.BlockSpec((1, n2), lambda i: (0, 0)),
            pl.BlockSpec((n2, n3), lambda i: (0, 0)),
            pl.BlockSpec((1, n3), lambda i: (0, 0)),
        ],
        out_specs=pl.BlockSpec((TB, n3), lambda i: (i, 0)),
        compiler_params=pltpu.CompilerParams(dimension_semantics=("parallel",)),
    )(flat, fc1_w.astype(bf16), fc1_b.astype(f32),
      fc2_w.astype(f32), fc2_b.astype(f32),
      fc3_w.astype(f32), fc3_b.astype(f32))
    return out[:N]
